# Initial kernel scaffold; baseline (speedup 1.0000x reference)
import jax
import jax.numpy as jnp
from jax.experimental import pallas as pl


def kernel(x_embed, prompt, prompt_key):
    raise NotImplementedError("write your pallas kernel here")


# speedup vs baseline: 1.7438x; 1.7438x over previous
import functools

import jax
import jax.numpy as jnp
from jax import lax
from jax.experimental import pallas as pl
from jax.experimental.pallas import tpu as pltpu
from jax.experimental.pallas import tpu_sc as plsc

POOL = 1000
KDIM = 3840
PDIM = 13824
BATCH = 1024
NTOK = 4
TOPK = 2

BCHUNK = 128
NBCH = BATCH // BCHUNK

NC = 2
NS = 16
NW = NC * NS
ROWS = BATCH * TOPK
RPW = ROWS // NW
CH = 8
NCHUNK = RPW // CH


def _pk_norm_body(pk_ref, out_ref):
    pk = pk_ref[...]
    sq = jnp.sum(pk * pk, axis=1, keepdims=True)
    out_ref[...] = pk * lax.rsqrt(jnp.maximum(sq, 1e-12))


def _sim_topk_body(x_ref, pkn_ref, sim_ref, i1_ref, i2_ref, acc_ref):
    x = x_ref[...]
    xm = jnp.mean(x, axis=1)
    sq = jnp.sum(xm * xm, axis=1, keepdims=True)
    xn = xm * lax.rsqrt(jnp.maximum(sq, 1e-12))
    pkn = pkn_ref[...]
    sim = lax.dot_general(xn, pkn, (((1,), (1,)), ((), ())),
                          preferred_element_type=jnp.float32)
    sim_ref[...] = sim
    col = lax.broadcasted_iota(jnp.int32, sim.shape, 1)
    big = jnp.int32(2**30)
    m1 = jnp.max(sim, axis=1, keepdims=True)
    i1 = jnp.min(jnp.where(sim == m1, col, big), axis=1, keepdims=True)
    sim2 = jnp.where(col == i1, -jnp.inf, sim)
    m2 = jnp.max(sim2, axis=1, keepdims=True)
    i2 = jnp.min(jnp.where(sim2 == m2, col, big), axis=1, keepdims=True)
    i1_ref[...] = i1
    i2_ref[...] = i2

    @pl.when(pl.program_id(0) == 0)
    def _():
        acc_ref[...] = jnp.zeros((1, 1), jnp.float32)

    acc_ref[...] = acc_ref[...] + (jnp.sum(m1) + jnp.sum(m2))


def _sim_topk(x_embed, pk_norm):
    return pl.pallas_call(
        _sim_topk_body,
        grid=(NBCH,),
        in_specs=[
            pl.BlockSpec((BCHUNK, NTOK, KDIM), lambda i: (i, 0, 0)),
            pl.BlockSpec((POOL, KDIM), lambda i: (0, 0)),
        ],
        out_specs=[
            pl.BlockSpec((BCHUNK, POOL), lambda i: (i, 0)),
            pl.BlockSpec((BCHUNK, 1), lambda i: (i, 0)),
            pl.BlockSpec((BCHUNK, 1), lambda i: (i, 0)),
            pl.BlockSpec((1, 1), lambda i: (0, 0)),
        ],
        out_shape=[
            jax.ShapeDtypeStruct((BATCH, POOL), jnp.float32),
            jax.ShapeDtypeStruct((BATCH, 1), jnp.int32),
            jax.ShapeDtypeStruct((BATCH, 1), jnp.int32),
            jax.ShapeDtypeStruct((1, 1), jnp.float32),
        ],
    )(x_embed, pk_norm)


def _pk_norm(prompt_key):
    return pl.pallas_call(
        _pk_norm_body,
        grid=(5,),
        in_specs=[pl.BlockSpec((200, KDIM), lambda i: (i, 0))],
        out_specs=pl.BlockSpec((200, KDIM), lambda i: (i, 0)),
        out_shape=jax.ShapeDtypeStruct((POOL, KDIM), jnp.float32),
    )(prompt_key)


@functools.cache
def _gather_kernel():
    mesh = plsc.VectorSubcoreMesh(core_axis_name="c", subcore_axis_name="s")

    @functools.partial(
        pl.kernel,
        mesh=mesh,
        out_type=jax.ShapeDtypeStruct((ROWS, PDIM), jnp.float32),
        scratch_types=[
            pltpu.VMEM((RPW,), jnp.int32),
            pltpu.VMEM((CH, PDIM), jnp.float32),
            pltpu.SemaphoreType.DMA,
        ],
    )
    def _gather_rows(prompt_hbm, idx_hbm, out_hbm, idx_v, buf, sem):
        wid = lax.axis_index("s") * NC + lax.axis_index("c")
        base = wid * RPW
        pltpu.sync_copy(idx_hbm.at[pl.ds(base, RPW)], idx_v)
        for c in range(NCHUNK):
            pltpu.async_copy(
                prompt_hbm.at[idx_v.at[pl.ds(c * CH, CH)]], buf, sem
            ).wait()
            pltpu.sync_copy(buf, out_hbm.at[pl.ds(base + c * CH, CH)])

    return _gather_rows


def kernel(x_embed, prompt, prompt_key):
    pk_norm = _pk_norm(prompt_key)
    sim, i1, i2, acc = _sim_topk(x_embed, pk_norm)
    idx = jnp.concatenate([i1, i2], axis=1)
    rows = _gather_kernel()(prompt.reshape(POOL, PDIM), idx.reshape(ROWS))
    batched_prompt = rows.reshape(BATCH, TOPK, PDIM)
    reduce_sim = acc[0, 0] / BATCH
    return batched_prompt, sim, idx, reduce_sim


# speedup vs baseline: 4.2144x; 2.4168x over previous
import functools

import jax
import jax.numpy as jnp
from jax import lax
from jax.experimental import pallas as pl
from jax.experimental.pallas import tpu as pltpu
from jax.experimental.pallas import tpu_sc as plsc

POOL = 1000
KDIM = 3840
PDIM = 13824
BATCH = 1024
NTOK = 4
TOPK = 2

BCHUNK = 128
NBCH = BATCH // BCHUNK

NC = 2
NS = 16
NW = NC * NS
ROWS = BATCH * TOPK
RPW = ROWS // NW
CH = 8
NCHUNK = RPW // CH


def _pk_norm_body(pk_ref, out_ref):
    pk = pk_ref[...]
    sq = jnp.sum(pk * pk, axis=1, keepdims=True)
    out_ref[...] = pk * lax.rsqrt(jnp.maximum(sq, 1e-12))


def _sim_topk_body(x_ref, pkn_ref, sim_ref, i1_ref, i2_ref, acc_ref):
    x = x_ref[...]
    xm = jnp.mean(x, axis=1)
    sq = jnp.sum(xm * xm, axis=1, keepdims=True)
    xn = xm * lax.rsqrt(jnp.maximum(sq, 1e-12))
    pkn = pkn_ref[...]
    sim = lax.dot_general(xn, pkn, (((1,), (1,)), ((), ())),
                          preferred_element_type=jnp.float32)
    sim_ref[...] = sim
    col = lax.broadcasted_iota(jnp.int32, sim.shape, 1)
    big = jnp.int32(2**30)
    m1 = jnp.max(sim, axis=1, keepdims=True)
    i1 = jnp.min(jnp.where(sim == m1, col, big), axis=1, keepdims=True)
    sim2 = jnp.where(col == i1, -jnp.inf, sim)
    m2 = jnp.max(sim2, axis=1, keepdims=True)
    i2 = jnp.min(jnp.where(sim2 == m2, col, big), axis=1, keepdims=True)
    i1_ref[...] = i1
    i2_ref[...] = i2

    @pl.when(pl.program_id(0) == 0)
    def _():
        acc_ref[...] = jnp.zeros((1, 1), jnp.float32)

    acc_ref[...] = acc_ref[...] + (jnp.sum(m1) + jnp.sum(m2))


def _sim_topk(x_embed, pk_norm):
    return pl.pallas_call(
        _sim_topk_body,
        grid=(NBCH,),
        in_specs=[
            pl.BlockSpec((BCHUNK, NTOK, KDIM), lambda i: (i, 0, 0)),
            pl.BlockSpec((POOL, KDIM), lambda i: (0, 0)),
        ],
        out_specs=[
            pl.BlockSpec((BCHUNK, POOL), lambda i: (i, 0)),
            pl.BlockSpec((BCHUNK, 1), lambda i: (i, 0)),
            pl.BlockSpec((BCHUNK, 1), lambda i: (i, 0)),
            pl.BlockSpec((1, 1), lambda i: (0, 0)),
        ],
        out_shape=[
            jax.ShapeDtypeStruct((BATCH, POOL), jnp.float32),
            jax.ShapeDtypeStruct((BATCH, 1), jnp.int32),
            jax.ShapeDtypeStruct((BATCH, 1), jnp.int32),
            jax.ShapeDtypeStruct((1, 1), jnp.float32),
        ],
    )(x_embed, pk_norm)


def _pk_norm(prompt_key):
    return pl.pallas_call(
        _pk_norm_body,
        grid=(5,),
        in_specs=[pl.BlockSpec((200, KDIM), lambda i: (i, 0))],
        out_specs=pl.BlockSpec((200, KDIM), lambda i: (i, 0)),
        out_shape=jax.ShapeDtypeStruct((POOL, KDIM), jnp.float32),
    )(prompt_key)


BPW = BATCH // NW
BCH = 4
NBC = BPW // BCH
NJOB = NBC * TOPK


@functools.cache
def _gather_kernel():
    mesh = plsc.VectorSubcoreMesh(core_axis_name="c", subcore_axis_name="s")

    @functools.partial(
        pl.kernel,
        mesh=mesh,
        out_type=jax.ShapeDtypeStruct((BATCH, TOPK, PDIM), jnp.float32),
        scratch_types=[
            pltpu.VMEM((NJOB, 8), jnp.int32),
            pltpu.VMEM((BCH, 1, PDIM), jnp.float32),
            pltpu.VMEM((BCH, 1, PDIM), jnp.float32),
            pltpu.SemaphoreType.DMA,
            pltpu.SemaphoreType.DMA,
            pltpu.SemaphoreType.DMA,
            pltpu.SemaphoreType.DMA,
        ],
    )
    def _gather_rows(prompt_hbm, ijobs_hbm, out_hbm,
                     idx_v, buf_a, buf_b,
                     gsem_a, gsem_b, wsem_a, wsem_b):
        wid = lax.axis_index("s") * NC + lax.axis_index("c")
        pltpu.sync_copy(ijobs_hbm.at[pl.ds(wid * NJOB, NJOB)], idx_v)
        writes = []
        for j in range(NJOB):
            c, p = j // 2, j % 2
            buf, gsem, wsem = (
                (buf_a, gsem_a, wsem_a) if j % 2 == 0 else (buf_b, gsem_b, wsem_b)
            )
            if j >= 2:
                writes[j - 2].wait()
            pltpu.async_copy(
                prompt_hbm.at[idx_v.at[j, pl.ds(0, BCH)]], buf, gsem
            ).wait()
            writes.append(
                pltpu.async_copy(
                    buf,
                    out_hbm.at[pl.ds(wid * BPW + c * BCH, BCH), pl.ds(p, 1)],
                    wsem,
                )
            )
        writes[-2].wait()
        writes[-1].wait()

    return _gather_rows


def kernel(x_embed, prompt, prompt_key):
    pk_norm = _pk_norm(prompt_key)
    sim, i1, i2, acc = _sim_topk(x_embed, pk_norm)
    idx = jnp.concatenate([i1, i2], axis=1)
    ijk = jnp.stack(
        [i1.reshape(NW, NBC, BCH), i2.reshape(NW, NBC, BCH)], axis=2
    )
    ijobs = jnp.concatenate([ijk, jnp.zeros_like(ijk)], axis=-1).reshape(
        NW * NJOB, 8
    )
    batched_prompt = _gather_kernel()(prompt, ijobs)
    reduce_sim = acc[0, 0] / BATCH
    return batched_prompt, sim, idx, reduce_sim
